# Initial kernel scaffold; baseline (speedup 1.0000x reference)
import jax
import jax.numpy as jnp
from jax.experimental import pallas as pl


def kernel(x, edge_index_x, ptr_x, y, edge_index_y, ptr_y, emb, W_skip, b_skip, W0, b0, W1, b1, ln0_g, ln0_b, ln1_g, ln1_b):
    raise NotImplementedError("write your pallas kernel here")


# speedup vs baseline: 6.6988x; 6.6988x over previous
import functools

import jax
import jax.numpy as jnp
from jax import lax
from jax.experimental import pallas as pl
from jax.experimental.pallas import tpu as pltpu
from jax.experimental.pallas import tpu_sc as plsc

N = 10000
D = 128
E = 320000
NB = 8

NC = 2
NS = 16
NW = NC * NS
L = 16

NPAD = 10240
RPT = NPAD // NW
ROWS_PER_SC_TILE = NPAD // NS

CH = 128
EPT = -(-E // NW)
NCHUNK = 80
EPT_PAD = NCHUNK * CH
EPAD = EPT_PAD * NW
DUMP_ROW = N + 200


def _sc_deg_gather_body(dst1d, xids, emb, z128, ones_h, deg_out, ex_out,
                        didx_cur, xidx, ones_v, erows, deg_acc, sem):
    cid = lax.axis_index("c")
    sid = lax.axis_index("s")
    wid = sid * NC + cid
    row0 = sid * ROWS_PER_SC_TILE

    pltpu.sync_copy(ones_h, ones_v)

    pltpu.sync_copy(z128.at[pl.ds(row0, ROWS_PER_SC_TILE)],
                    deg_acc.at[pl.ds(row0, ROWS_PER_SC_TILE)])
    plsc.subcore_barrier()

    ebase = wid * EPT_PAD
    for j in range(NCHUNK):
        pltpu.sync_copy(dst1d.at[pl.ds(ebase + j * CH, CH)], didx_cur)
        pltpu.async_copy(ones_v, deg_acc.at[didx_cur], sem, add=True).wait()

    pltpu.sync_copy(xids.at[pl.ds(wid * RPT, RPT)], xidx)
    for j in range(RPT // 64):
        pltpu.async_copy(emb.at[xidx.at[pl.ds(j * 64, 64)]], erows, sem).wait()
        pltpu.sync_copy(erows, ex_out.at[pl.ds(wid * RPT + j * 64, 64)])

    plsc.subcore_barrier()
    pltpu.sync_copy(deg_acc.at[pl.ds(row0, ROWS_PER_SC_TILE)],
                    deg_out.at[cid, pl.ds(row0, ROWS_PER_SC_TILE)])


def _sc_edge_agg_body(hs, src1d, dst1d, z128, agg_out, sidx_cur, didx_cur,
                      rows, acc, sem):
    cid = lax.axis_index("c")
    sid = lax.axis_index("s")
    wid = sid * NC + cid

    row0 = sid * ROWS_PER_SC_TILE
    pltpu.sync_copy(z128.at[pl.ds(row0, ROWS_PER_SC_TILE)],
                    acc.at[pl.ds(row0, ROWS_PER_SC_TILE)])
    plsc.subcore_barrier()

    ebase = wid * EPT_PAD

    def chunk(j, _):
        pltpu.sync_copy(src1d.at[pl.ds(ebase + j * CH, CH)], sidx_cur)
        pltpu.sync_copy(dst1d.at[pl.ds(ebase + j * CH, CH)], didx_cur)
        pltpu.async_copy(hs.at[sidx_cur], rows, sem).wait()
        pltpu.sync_copy(rows, acc.at[didx_cur], add=True)
        return 0

    lax.fori_loop(0, NCHUNK, chunk, 0)

    plsc.subcore_barrier()
    pltpu.sync_copy(acc.at[pl.ds(row0, ROWS_PER_SC_TILE)],
                    agg_out.at[cid, pl.ds(row0, ROWS_PER_SC_TILE)])


@functools.cache
def _sc_kernels():
    mesh = plsc.VectorSubcoreMesh(core_axis_name="c", subcore_axis_name="s",
                                  num_cores=NC, num_subcores=NS)
    deg_gather = pl.kernel(
        _sc_deg_gather_body,
        out_type=(
            jax.ShapeDtypeStruct((NC, NPAD, D), jnp.float32),
            jax.ShapeDtypeStruct((NPAD, D), jnp.float32),
        ),
        mesh=mesh,
        scratch_types=[
            pltpu.VMEM((CH,), jnp.int32),
            pltpu.VMEM((RPT,), jnp.int32),
            pltpu.VMEM((CH, D), jnp.float32),
            pltpu.VMEM((64, D), jnp.float32),
            pltpu.VMEM_SHARED((NPAD, D), jnp.float32),
            pltpu.SemaphoreType.DMA,
        ],
    )
    edge_agg = pl.kernel(
        _sc_edge_agg_body,
        out_type=jax.ShapeDtypeStruct((NC, NPAD, D), jnp.float32),
        mesh=mesh,
        scratch_types=[
            pltpu.VMEM((CH,), jnp.int32),
            pltpu.VMEM((CH,), jnp.int32),
            pltpu.VMEM((CH, D), jnp.float32),
            pltpu.VMEM_SHARED((NPAD, D), jnp.float32),
            pltpu.SemaphoreType.DMA,
        ],
    )
    return deg_gather, edge_agg


BR = 512
GRID = NPAD // BR


def _dinv_from(degp):
    deg = degp[0, :, 0] + degp[1, :, 0] + 1.0
    return lax.rsqrt(deg)[:, None]


def _tc1_body(ex_ref, degp_ref, w0_ref, wskip_ref, bskip_ref, hs1_ref, skip_ref):
    ex = ex_ref[...]
    dinv = _dinv_from(degp_ref[...])
    h2 = jnp.dot(ex, w0_ref[...], preferred_element_type=jnp.float32)
    hs1_ref[...] = h2 * dinv
    skip_ref[...] = (jnp.dot(ex, wskip_ref[...], preferred_element_type=jnp.float32)
                     + bskip_ref[...])


def _layer_norm(h, g, b):
    mu = jnp.mean(h, axis=-1, keepdims=True)
    var = jnp.mean((h - mu) * (h - mu), axis=-1, keepdims=True)
    return (h - mu) * lax.rsqrt(var + 1e-5) * g + b


def _tc2_body(aggp_ref, hs1_ref, degp_ref, skip_ref, w1_ref, b0_ref,
              g0_ref, be0_ref, hs2_ref):
    dinv = _dinv_from(degp_ref[...])
    aggp = aggp_ref[...]
    out = (aggp[0] + aggp[1] + hs1_ref[...]) * dinv + b0_ref[...]
    h = jax.nn.relu(_layer_norm(out, g0_ref[...], be0_ref[...]))
    u = skip_ref[...] + h
    h2b = jnp.dot(u, w1_ref[...], preferred_element_type=jnp.float32)
    hs2_ref[...] = h2b * dinv


def _tc3_body(aggp_ref, hs2_ref, degp_ref, b1_ref, g1_ref, be1_ref, ptr_ref,
              h2_ref, g_ref):
    i = pl.program_id(0)
    dinv = _dinv_from(degp_ref[...])
    aggp = aggp_ref[...]
    out = (aggp[0] + aggp[1] + hs2_ref[...]) * dinv + b1_ref[...]
    h2 = jax.nn.relu(_layer_norm(out, g1_ref[...], be1_ref[...]))
    h2_ref[...] = h2

    rows = lax.broadcasted_iota(jnp.int32, (NB, BR), 1) + i * BR
    lo = jnp.stack([ptr_ref[b] for b in range(NB)])[:, None]
    hi = jnp.stack([ptr_ref[b + 1] for b in range(NB)])[:, None]
    cnt = jnp.maximum((hi - lo).astype(jnp.float32), 1.0)
    p = jnp.where((rows >= lo) & (rows < hi), 1.0, 0.0) / cnt
    gp = jnp.dot(p, h2, preferred_element_type=jnp.float32)

    @pl.when(i == 0)
    def _():
        g_ref[...] = jnp.zeros_like(g_ref)

    g_ref[...] += gp


def _row_spec(i_map=lambda i: (i, 0)):
    return pl.BlockSpec((BR, D), i_map)


_full128 = pl.BlockSpec((D, D), lambda i: (0, 0))
_vec128 = pl.BlockSpec((1, D), lambda i: (0, 0))
_degp_spec = pl.BlockSpec((NC, BR, D), lambda i: (0, i, 0))
_aggp_spec = pl.BlockSpec((NC, BR, D), lambda i: (0, i, 0))


def _tc1(ex, degp, w0, wskip, bskip):
    return pl.pallas_call(
        _tc1_body,
        grid=(GRID,),
        in_specs=[_row_spec(), _degp_spec, _full128, _full128, _vec128],
        out_specs=[_row_spec(), _row_spec()],
        out_shape=[jax.ShapeDtypeStruct((NPAD, D), jnp.float32)] * 2,
    )(ex, degp, w0, wskip, bskip)


def _tc2(aggp, hs1, degp, skip, w1, b0, g0, be0):
    return pl.pallas_call(
        _tc2_body,
        grid=(GRID,),
        in_specs=[_aggp_spec, _row_spec(), _degp_spec, _row_spec(), _full128,
                  _vec128, _vec128, _vec128],
        out_specs=_row_spec(),
        out_shape=jax.ShapeDtypeStruct((NPAD, D), jnp.float32),
    )(aggp, hs1, degp, skip, w1, b0, g0, be0)


def _tc3(aggp, hs2, degp, b1, g1, be1, ptr):
    return pl.pallas_call(
        _tc3_body,
        grid=(GRID,),
        in_specs=[_aggp_spec, _row_spec(), _degp_spec, _vec128, _vec128,
                  _vec128, pl.BlockSpec(memory_space=pltpu.SMEM)],
        out_specs=[_row_spec(), pl.BlockSpec((NB, D), lambda i: (0, 0))],
        out_shape=[jax.ShapeDtypeStruct((NPAD, D), jnp.float32),
                   jax.ShapeDtypeStruct((NB, D), jnp.float32)],
    )(aggp, hs2, degp, b1, g1, be1, ptr)


def kernel(x, edge_index_x, ptr_x, y, edge_index_y, ptr_y, emb,
           W_skip, b_skip, W0, b0, W1, b1, ln0_g, ln0_b, ln1_g, ln1_b):
    src = edge_index_x[0]
    dst = edge_index_x[1]
    pad_e = EPAD - E
    src1d = jnp.concatenate([src, jnp.zeros((pad_e,), jnp.int32)])
    dst1d = jnp.concatenate([dst, jnp.full((pad_e,), DUMP_ROW, jnp.int32)])
    xids = jnp.concatenate([x, jnp.zeros((NPAD - N,), jnp.int32)])

    _sc_deg_gather, _sc_edge_agg = _sc_kernels()
    z128 = jnp.zeros((NPAD, D), jnp.float32)
    ones_h = jnp.zeros((CH, D), jnp.float32).at[:, 0].set(1.0)
    degp, ex = _sc_deg_gather(dst1d, xids, emb, z128, ones_h)

    hs1, skip = _tc1(ex, degp, W0, W_skip, b_skip.reshape(1, D))
    agg1 = _sc_edge_agg(hs1, src1d, dst1d, z128)
    hs2 = _tc2(agg1, hs1, degp, skip, W1, b0.reshape(1, D),
               ln0_g.reshape(1, D), ln0_b.reshape(1, D))
    agg2 = _sc_edge_agg(hs2, src1d, dst1d, z128)
    h2pad, g = _tc3(agg2, hs2, degp, b1.reshape(1, D),
                    ln1_g.reshape(1, D), ln1_b.reshape(1, D), ptr_x)
    return (h2pad[:N], g)


# speedup vs baseline: 8.1038x; 1.2097x over previous
import functools

import jax
import jax.numpy as jnp
from jax import lax
from jax.experimental import pallas as pl
from jax.experimental.pallas import tpu as pltpu
from jax.experimental.pallas import tpu_sc as plsc

N = 10000
D = 128
E = 320000
NB = 8

NC = 2
NS = 16
NW = NC * NS
L = 16

NPAD = 10240
RPT = NPAD // NW
ROWS_PER_SC_TILE = NPAD // NS

CH = 128
EPT = -(-E // NW)
NCHUNK = 80
EPT_PAD = NCHUNK * CH
EPAD = EPT_PAD * NW
DUMP_ROW = N + 200


def _sc_deg_gather_body(dst1d, xids, emb, z128, ones_h, deg_out, ex_out,
                        didx_cur, xidx, ones_v, erows, deg_acc, sem):
    cid = lax.axis_index("c")
    sid = lax.axis_index("s")
    wid = sid * NC + cid
    row0 = sid * ROWS_PER_SC_TILE

    pltpu.sync_copy(ones_h, ones_v)

    pltpu.sync_copy(z128.at[pl.ds(row0, ROWS_PER_SC_TILE)],
                    deg_acc.at[pl.ds(row0, ROWS_PER_SC_TILE)])
    plsc.subcore_barrier()

    ebase = wid * EPT_PAD
    for j in range(NCHUNK):
        pltpu.sync_copy(dst1d.at[pl.ds(ebase + j * CH, CH)], didx_cur)
        pltpu.async_copy(ones_v, deg_acc.at[didx_cur], sem, add=True).wait()

    pltpu.sync_copy(xids.at[pl.ds(wid * RPT, RPT)], xidx)
    for j in range(RPT // 64):
        pltpu.async_copy(emb.at[xidx.at[pl.ds(j * 64, 64)]], erows, sem).wait()
        pltpu.sync_copy(erows, ex_out.at[pl.ds(wid * RPT + j * 64, 64)])

    plsc.subcore_barrier()
    pltpu.sync_copy(deg_acc.at[pl.ds(row0, ROWS_PER_SC_TILE)],
                    deg_out.at[cid, pl.ds(row0, ROWS_PER_SC_TILE)])


def _sc_edge_agg_body(hs, src1d, dst1d, z128, agg_out, sidx0, sidx1, didx0,
                      didx1, rows0, rows1, acc, sem0, sem1):
    cid = lax.axis_index("c")
    sid = lax.axis_index("s")
    wid = sid * NC + cid

    row0 = sid * ROWS_PER_SC_TILE
    pltpu.sync_copy(z128.at[pl.ds(row0, ROWS_PER_SC_TILE)],
                    acc.at[pl.ds(row0, ROWS_PER_SC_TILE)])
    plsc.subcore_barrier()

    ebase = wid * EPT_PAD
    sidx = (sidx0, sidx1)
    didx = (didx0, didx1)
    rows = (rows0, rows1)
    sems = (sem0, sem1)

    pltpu.sync_copy(src1d.at[pl.ds(ebase, CH)], sidx[0])
    pltpu.sync_copy(dst1d.at[pl.ds(ebase, CH)], didx[0])
    gather = [None, None]
    gather[0] = pltpu.async_copy(hs.at[sidx[0]], rows[0], sems[0])

    for j in range(NCHUNK):
        b = j % 2
        nb = (j + 1) % 2
        if j + 1 < NCHUNK:
            off = ebase + (j + 1) * CH
            pltpu.sync_copy(src1d.at[pl.ds(off, CH)], sidx[nb])
            pltpu.sync_copy(dst1d.at[pl.ds(off, CH)], didx[nb])
            gather[nb] = pltpu.async_copy(hs.at[sidx[nb]], rows[nb], sems[nb])
        gather[b].wait()
        pltpu.sync_copy(rows[b], acc.at[didx[b]], add=True)

    plsc.subcore_barrier()
    pltpu.sync_copy(acc.at[pl.ds(row0, ROWS_PER_SC_TILE)],
                    agg_out.at[cid, pl.ds(row0, ROWS_PER_SC_TILE)])


@functools.cache
def _sc_kernels():
    mesh = plsc.VectorSubcoreMesh(core_axis_name="c", subcore_axis_name="s",
                                  num_cores=NC, num_subcores=NS)
    deg_gather = pl.kernel(
        _sc_deg_gather_body,
        out_type=(
            jax.ShapeDtypeStruct((NC, NPAD, D), jnp.float32),
            jax.ShapeDtypeStruct((NPAD, D), jnp.float32),
        ),
        mesh=mesh,
        scratch_types=[
            pltpu.VMEM((CH,), jnp.int32),
            pltpu.VMEM((RPT,), jnp.int32),
            pltpu.VMEM((CH, D), jnp.float32),
            pltpu.VMEM((64, D), jnp.float32),
            pltpu.VMEM_SHARED((NPAD, D), jnp.float32),
            pltpu.SemaphoreType.DMA,
        ],
    )
    edge_agg = pl.kernel(
        _sc_edge_agg_body,
        out_type=jax.ShapeDtypeStruct((NC, NPAD, D), jnp.float32),
        mesh=mesh,
        scratch_types=[
            pltpu.VMEM((CH,), jnp.int32),
            pltpu.VMEM((CH,), jnp.int32),
            pltpu.VMEM((CH,), jnp.int32),
            pltpu.VMEM((CH,), jnp.int32),
            pltpu.VMEM((CH, D), jnp.float32),
            pltpu.VMEM((CH, D), jnp.float32),
            pltpu.VMEM_SHARED((NPAD, D), jnp.float32),
            pltpu.SemaphoreType.DMA,
            pltpu.SemaphoreType.DMA,
        ],
    )
    return deg_gather, edge_agg


BR = 512
GRID = NPAD // BR


def _dinv_from(degp):
    deg = degp[0, :, 0] + degp[1, :, 0] + 1.0
    return lax.rsqrt(deg)[:, None]


def _tc1_body(ex_ref, degp_ref, w0_ref, wskip_ref, bskip_ref, hs1_ref, skip_ref):
    ex = ex_ref[...]
    dinv = _dinv_from(degp_ref[...])
    h2 = jnp.dot(ex, w0_ref[...], preferred_element_type=jnp.float32)
    hs1_ref[...] = h2 * dinv
    skip_ref[...] = (jnp.dot(ex, wskip_ref[...], preferred_element_type=jnp.float32)
                     + bskip_ref[...])


def _layer_norm(h, g, b):
    mu = jnp.mean(h, axis=-1, keepdims=True)
    var = jnp.mean((h - mu) * (h - mu), axis=-1, keepdims=True)
    return (h - mu) * lax.rsqrt(var + 1e-5) * g + b


def _tc2_body(aggp_ref, hs1_ref, degp_ref, skip_ref, w1_ref, b0_ref,
              g0_ref, be0_ref, hs2_ref):
    dinv = _dinv_from(degp_ref[...])
    aggp = aggp_ref[...]
    out = (aggp[0] + aggp[1] + hs1_ref[...]) * dinv + b0_ref[...]
    h = jax.nn.relu(_layer_norm(out, g0_ref[...], be0_ref[...]))
    u = skip_ref[...] + h
    h2b = jnp.dot(u, w1_ref[...], preferred_element_type=jnp.float32)
    hs2_ref[...] = h2b * dinv


def _tc3_body(aggp_ref, hs2_ref, degp_ref, b1_ref, g1_ref, be1_ref, ptr_ref,
              h2_ref, g_ref):
    i = pl.program_id(0)
    dinv = _dinv_from(degp_ref[...])
    aggp = aggp_ref[...]
    out = (aggp[0] + aggp[1] + hs2_ref[...]) * dinv + b1_ref[...]
    h2 = jax.nn.relu(_layer_norm(out, g1_ref[...], be1_ref[...]))
    h2_ref[...] = h2

    rows = lax.broadcasted_iota(jnp.int32, (NB, BR), 1) + i * BR
    lo = jnp.stack([ptr_ref[b] for b in range(NB)])[:, None]
    hi = jnp.stack([ptr_ref[b + 1] for b in range(NB)])[:, None]
    cnt = jnp.maximum((hi - lo).astype(jnp.float32), 1.0)
    p = jnp.where((rows >= lo) & (rows < hi), 1.0, 0.0) / cnt
    gp = jnp.dot(p, h2, preferred_element_type=jnp.float32)

    @pl.when(i == 0)
    def _():
        g_ref[...] = jnp.zeros_like(g_ref)

    g_ref[...] += gp


def _row_spec(i_map=lambda i: (i, 0)):
    return pl.BlockSpec((BR, D), i_map)


_full128 = pl.BlockSpec((D, D), lambda i: (0, 0))
_vec128 = pl.BlockSpec((1, D), lambda i: (0, 0))
_degp_spec = pl.BlockSpec((NC, BR, D), lambda i: (0, i, 0))
_aggp_spec = pl.BlockSpec((NC, BR, D), lambda i: (0, i, 0))


def _tc1(ex, degp, w0, wskip, bskip):
    return pl.pallas_call(
        _tc1_body,
        grid=(GRID,),
        in_specs=[_row_spec(), _degp_spec, _full128, _full128, _vec128],
        out_specs=[_row_spec(), _row_spec()],
        out_shape=[jax.ShapeDtypeStruct((NPAD, D), jnp.float32)] * 2,
    )(ex, degp, w0, wskip, bskip)


def _tc2(aggp, hs1, degp, skip, w1, b0, g0, be0):
    return pl.pallas_call(
        _tc2_body,
        grid=(GRID,),
        in_specs=[_aggp_spec, _row_spec(), _degp_spec, _row_spec(), _full128,
                  _vec128, _vec128, _vec128],
        out_specs=_row_spec(),
        out_shape=jax.ShapeDtypeStruct((NPAD, D), jnp.float32),
    )(aggp, hs1, degp, skip, w1, b0, g0, be0)


def _tc3(aggp, hs2, degp, b1, g1, be1, ptr):
    return pl.pallas_call(
        _tc3_body,
        grid=(GRID,),
        in_specs=[_aggp_spec, _row_spec(), _degp_spec, _vec128, _vec128,
                  _vec128, pl.BlockSpec(memory_space=pltpu.SMEM)],
        out_specs=[_row_spec(), pl.BlockSpec((NB, D), lambda i: (0, 0))],
        out_shape=[jax.ShapeDtypeStruct((NPAD, D), jnp.float32),
                   jax.ShapeDtypeStruct((NB, D), jnp.float32)],
    )(aggp, hs2, degp, b1, g1, be1, ptr)


def kernel(x, edge_index_x, ptr_x, y, edge_index_y, ptr_y, emb,
           W_skip, b_skip, W0, b0, W1, b1, ln0_g, ln0_b, ln1_g, ln1_b):
    src = edge_index_x[0]
    dst = edge_index_x[1]
    pad_e = EPAD - E
    src1d = jnp.concatenate([src, jnp.zeros((pad_e,), jnp.int32)])
    dst1d = jnp.concatenate([dst, jnp.full((pad_e,), DUMP_ROW, jnp.int32)])
    xids = jnp.concatenate([x, jnp.zeros((NPAD - N,), jnp.int32)])

    _sc_deg_gather, _sc_edge_agg = _sc_kernels()
    z128 = jnp.zeros((NPAD, D), jnp.float32)
    ones_h = jnp.zeros((CH, D), jnp.float32).at[:, 0].set(1.0)
    degp, ex = _sc_deg_gather(dst1d, xids, emb, z128, ones_h)

    hs1, skip = _tc1(ex, degp, W0, W_skip, b_skip.reshape(1, D))
    agg1 = _sc_edge_agg(hs1, src1d, dst1d, z128)
    hs2 = _tc2(agg1, hs1, degp, skip, W1, b0.reshape(1, D),
               ln0_g.reshape(1, D), ln0_b.reshape(1, D))
    agg2 = _sc_edge_agg(hs2, src1d, dst1d, z128)
    h2pad, g = _tc3(agg2, hs2, degp, b1.reshape(1, D),
                    ln1_g.reshape(1, D), ln1_b.reshape(1, D), ptr_x)
    return (h2pad[:N], g)


# speedup vs baseline: 8.1480x; 1.0055x over previous
import functools

import jax
import jax.numpy as jnp
from jax import lax
from jax.experimental import pallas as pl
from jax.experimental.pallas import tpu as pltpu
from jax.experimental.pallas import tpu_sc as plsc

N = 10000
D = 128
E = 320000
NB = 8

NC = 2
NS = 16
NW = NC * NS
L = 16

NPAD = 10240
RPT = NPAD // NW
ROWS_PER_SC_TILE = NPAD // NS

CH = 128
EPT = -(-E // NW)
NCHUNK = 80
EPT_PAD = NCHUNK * CH
EPAD = EPT_PAD * NW
DUMP_ROW = N + 200


def _sc_deg_gather_body(dst1d, xids, emb, z128, ones_h, deg_out, ex_out,
                        didx_cur, xidx, ones_v, erows, deg_acc, sem):
    cid = lax.axis_index("c")
    sid = lax.axis_index("s")
    wid = sid * NC + cid
    row0 = sid * ROWS_PER_SC_TILE

    pltpu.sync_copy(ones_h, ones_v)

    pltpu.sync_copy(z128.at[pl.ds(row0, ROWS_PER_SC_TILE)],
                    deg_acc.at[pl.ds(row0, ROWS_PER_SC_TILE)])
    plsc.subcore_barrier()

    ebase = wid * EPT_PAD
    for j in range(NCHUNK):
        pltpu.sync_copy(dst1d.at[pl.ds(ebase + j * CH, CH)], didx_cur)
        pltpu.async_copy(ones_v, deg_acc.at[didx_cur], sem, add=True).wait()

    pltpu.sync_copy(xids.at[pl.ds(wid * RPT, RPT)], xidx)
    for j in range(RPT // 64):
        pltpu.async_copy(emb.at[xidx.at[pl.ds(j * 64, 64)]], erows, sem).wait()
        pltpu.sync_copy(erows, ex_out.at[pl.ds(wid * RPT + j * 64, 64)])

    plsc.subcore_barrier()
    pltpu.sync_copy(deg_acc.at[pl.ds(row0, ROWS_PER_SC_TILE)],
                    deg_out.at[cid, pl.ds(row0, ROWS_PER_SC_TILE)])


def _sc_edge_agg_body(hs, src1d, dst1d, z128, agg_out, sidx0, sidx1, didx0,
                      didx1, rows0, rows1, acc, sem0, sem1, isem0, isem1):
    cid = lax.axis_index("c")
    sid = lax.axis_index("s")
    wid = sid * NC + cid

    row0 = sid * ROWS_PER_SC_TILE
    pltpu.sync_copy(z128.at[pl.ds(row0, ROWS_PER_SC_TILE)],
                    acc.at[pl.ds(row0, ROWS_PER_SC_TILE)])
    plsc.subcore_barrier()

    ebase = wid * EPT_PAD
    sidx = (sidx0, sidx1)
    didx = (didx0, didx1)
    rows = (rows0, rows1)
    gsems = (sem0, sem1)
    isems = (isem0, isem1)

    pltpu.sync_copy(src1d.at[pl.ds(ebase, CH)], sidx[0])
    pltpu.sync_copy(dst1d.at[pl.ds(ebase, CH)], didx[0])
    gather = [None, None]
    idxs = [None, None]
    idxd = [None, None]
    gather[0] = pltpu.async_copy(hs.at[sidx[0]], rows[0], gsems[0])
    if NCHUNK > 1:
        idxs[1] = pltpu.async_copy(src1d.at[pl.ds(ebase + CH, CH)], sidx[1],
                                   isems[1])
        idxd[1] = pltpu.async_copy(dst1d.at[pl.ds(ebase + CH, CH)], didx[1],
                                   isems[1])

    for j in range(NCHUNK):
        b = j % 2
        nb = (j + 1) % 2
        if j + 1 < NCHUNK:
            idxs[nb].wait()
            idxd[nb].wait()
            gather[nb] = pltpu.async_copy(hs.at[sidx[nb]], rows[nb], gsems[nb])
        gather[b].wait()
        pltpu.sync_copy(rows[b], acc.at[didx[b]], add=True)
        if j + 2 < NCHUNK:
            off = ebase + (j + 2) * CH
            idxs[b] = pltpu.async_copy(src1d.at[pl.ds(off, CH)], sidx[b],
                                       isems[b])
            idxd[b] = pltpu.async_copy(dst1d.at[pl.ds(off, CH)], didx[b],
                                       isems[b])

    plsc.subcore_barrier()
    pltpu.sync_copy(acc.at[pl.ds(row0, ROWS_PER_SC_TILE)],
                    agg_out.at[cid, pl.ds(row0, ROWS_PER_SC_TILE)])


@functools.cache
def _sc_kernels():
    mesh = plsc.VectorSubcoreMesh(core_axis_name="c", subcore_axis_name="s",
                                  num_cores=NC, num_subcores=NS)
    deg_gather = pl.kernel(
        _sc_deg_gather_body,
        out_type=(
            jax.ShapeDtypeStruct((NC, NPAD, D), jnp.float32),
            jax.ShapeDtypeStruct((NPAD, D), jnp.float32),
        ),
        mesh=mesh,
        scratch_types=[
            pltpu.VMEM((CH,), jnp.int32),
            pltpu.VMEM((RPT,), jnp.int32),
            pltpu.VMEM((CH, D), jnp.float32),
            pltpu.VMEM((64, D), jnp.float32),
            pltpu.VMEM_SHARED((NPAD, D), jnp.float32),
            pltpu.SemaphoreType.DMA,
        ],
    )
    edge_agg = pl.kernel(
        _sc_edge_agg_body,
        out_type=jax.ShapeDtypeStruct((NC, NPAD, D), jnp.float32),
        mesh=mesh,
        scratch_types=[
            pltpu.VMEM((CH,), jnp.int32),
            pltpu.VMEM((CH,), jnp.int32),
            pltpu.VMEM((CH,), jnp.int32),
            pltpu.VMEM((CH,), jnp.int32),
            pltpu.VMEM((CH, D), jnp.float32),
            pltpu.VMEM((CH, D), jnp.float32),
            pltpu.VMEM_SHARED((NPAD, D), jnp.float32),
            pltpu.SemaphoreType.DMA,
            pltpu.SemaphoreType.DMA,
            pltpu.SemaphoreType.DMA,
            pltpu.SemaphoreType.DMA,
        ],
    )
    return deg_gather, edge_agg


BR = 512
GRID = NPAD // BR


def _dinv_from(degp):
    deg = degp[0, :, 0] + degp[1, :, 0] + 1.0
    return lax.rsqrt(deg)[:, None]


def _tc1_body(ex_ref, degp_ref, w0_ref, wskip_ref, bskip_ref, hs1_ref, skip_ref):
    ex = ex_ref[...]
    dinv = _dinv_from(degp_ref[...])
    h2 = jnp.dot(ex, w0_ref[...], preferred_element_type=jnp.float32)
    hs1_ref[...] = h2 * dinv
    skip_ref[...] = (jnp.dot(ex, wskip_ref[...], preferred_element_type=jnp.float32)
                     + bskip_ref[...])


def _layer_norm(h, g, b):
    mu = jnp.mean(h, axis=-1, keepdims=True)
    var = jnp.mean((h - mu) * (h - mu), axis=-1, keepdims=True)
    return (h - mu) * lax.rsqrt(var + 1e-5) * g + b


def _tc2_body(aggp_ref, hs1_ref, degp_ref, skip_ref, w1_ref, b0_ref,
              g0_ref, be0_ref, hs2_ref):
    dinv = _dinv_from(degp_ref[...])
    aggp = aggp_ref[...]
    out = (aggp[0] + aggp[1] + hs1_ref[...]) * dinv + b0_ref[...]
    h = jax.nn.relu(_layer_norm(out, g0_ref[...], be0_ref[...]))
    u = skip_ref[...] + h
    h2b = jnp.dot(u, w1_ref[...], preferred_element_type=jnp.float32)
    hs2_ref[...] = h2b * dinv


def _tc3_body(aggp_ref, hs2_ref, degp_ref, b1_ref, g1_ref, be1_ref, ptr_ref,
              h2_ref, g_ref):
    i = pl.program_id(0)
    dinv = _dinv_from(degp_ref[...])
    aggp = aggp_ref[...]
    out = (aggp[0] + aggp[1] + hs2_ref[...]) * dinv + b1_ref[...]
    h2 = jax.nn.relu(_layer_norm(out, g1_ref[...], be1_ref[...]))
    h2_ref[...] = h2

    rows = lax.broadcasted_iota(jnp.int32, (NB, BR), 1) + i * BR
    lo = jnp.stack([ptr_ref[b] for b in range(NB)])[:, None]
    hi = jnp.stack([ptr_ref[b + 1] for b in range(NB)])[:, None]
    cnt = jnp.maximum((hi - lo).astype(jnp.float32), 1.0)
    p = jnp.where((rows >= lo) & (rows < hi), 1.0, 0.0) / cnt
    gp = jnp.dot(p, h2, preferred_element_type=jnp.float32)

    @pl.when(i == 0)
    def _():
        g_ref[...] = jnp.zeros_like(g_ref)

    g_ref[...] += gp


def _row_spec(i_map=lambda i: (i, 0)):
    return pl.BlockSpec((BR, D), i_map)


_full128 = pl.BlockSpec((D, D), lambda i: (0, 0))
_vec128 = pl.BlockSpec((1, D), lambda i: (0, 0))
_degp_spec = pl.BlockSpec((NC, BR, D), lambda i: (0, i, 0))
_aggp_spec = pl.BlockSpec((NC, BR, D), lambda i: (0, i, 0))


def _tc1(ex, degp, w0, wskip, bskip):
    return pl.pallas_call(
        _tc1_body,
        grid=(GRID,),
        in_specs=[_row_spec(), _degp_spec, _full128, _full128, _vec128],
        out_specs=[_row_spec(), _row_spec()],
        out_shape=[jax.ShapeDtypeStruct((NPAD, D), jnp.float32)] * 2,
    )(ex, degp, w0, wskip, bskip)


def _tc2(aggp, hs1, degp, skip, w1, b0, g0, be0):
    return pl.pallas_call(
        _tc2_body,
        grid=(GRID,),
        in_specs=[_aggp_spec, _row_spec(), _degp_spec, _row_spec(), _full128,
                  _vec128, _vec128, _vec128],
        out_specs=_row_spec(),
        out_shape=jax.ShapeDtypeStruct((NPAD, D), jnp.float32),
    )(aggp, hs1, degp, skip, w1, b0, g0, be0)


def _tc3(aggp, hs2, degp, b1, g1, be1, ptr):
    return pl.pallas_call(
        _tc3_body,
        grid=(GRID,),
        in_specs=[_aggp_spec, _row_spec(), _degp_spec, _vec128, _vec128,
                  _vec128, pl.BlockSpec(memory_space=pltpu.SMEM)],
        out_specs=[_row_spec(), pl.BlockSpec((NB, D), lambda i: (0, 0))],
        out_shape=[jax.ShapeDtypeStruct((NPAD, D), jnp.float32),
                   jax.ShapeDtypeStruct((NB, D), jnp.float32)],
    )(aggp, hs2, degp, b1, g1, be1, ptr)


def kernel(x, edge_index_x, ptr_x, y, edge_index_y, ptr_y, emb,
           W_skip, b_skip, W0, b0, W1, b1, ln0_g, ln0_b, ln1_g, ln1_b):
    src = edge_index_x[0]
    dst = edge_index_x[1]
    pad_e = EPAD - E
    src1d = jnp.concatenate([src, jnp.zeros((pad_e,), jnp.int32)])
    dst1d = jnp.concatenate([dst, jnp.full((pad_e,), DUMP_ROW, jnp.int32)])
    xids = jnp.concatenate([x, jnp.zeros((NPAD - N,), jnp.int32)])

    _sc_deg_gather, _sc_edge_agg = _sc_kernels()
    z128 = jnp.zeros((NPAD, D), jnp.float32)
    ones_h = jnp.zeros((CH, D), jnp.float32).at[:, 0].set(1.0)
    degp, ex = _sc_deg_gather(dst1d, xids, emb, z128, ones_h)

    hs1, skip = _tc1(ex, degp, W0, W_skip, b_skip.reshape(1, D))
    agg1 = _sc_edge_agg(hs1, src1d, dst1d, z128)
    hs2 = _tc2(agg1, hs1, degp, skip, W1, b0.reshape(1, D),
               ln0_g.reshape(1, D), ln0_b.reshape(1, D))
    agg2 = _sc_edge_agg(hs2, src1d, dst1d, z128)
    h2pad, g = _tc3(agg2, hs2, degp, b1.reshape(1, D),
                    ln1_g.reshape(1, D), ln1_b.reshape(1, D), ptr_x)
    return (h2pad[:N], g)


# speedup vs baseline: 8.1733x; 1.0031x over previous
import functools

import jax
import jax.numpy as jnp
from jax import lax
from jax.experimental import pallas as pl
from jax.experimental.pallas import tpu as pltpu
from jax.experimental.pallas import tpu_sc as plsc

N = 10000
D = 128
E = 320000
NB = 8

NC = 2
NS = 16
NW = NC * NS
L = 16

NPAD = 10240
RPT = NPAD // NW
ROWS_PER_SC_TILE = NPAD // NS

CH = 128
EPT = -(-E // NW)
NCHUNK = 80
EPT_PAD = NCHUNK * CH
EPAD = EPT_PAD * NW
DUMP_ROW = N + 200


def _sc_deg_gather_body(dst1d, xids, emb, z128, ones_h, deg_out, ex_out,
                        didx_cur, xidx, ones_v, erows, deg_acc, sem):
    cid = lax.axis_index("c")
    sid = lax.axis_index("s")
    wid = sid * NC + cid
    row0 = sid * ROWS_PER_SC_TILE

    pltpu.sync_copy(ones_h, ones_v)

    pltpu.sync_copy(z128.at[pl.ds(row0, ROWS_PER_SC_TILE)],
                    deg_acc.at[pl.ds(row0, ROWS_PER_SC_TILE)])
    plsc.subcore_barrier()

    ebase = wid * EPT_PAD
    for j in range(NCHUNK):
        pltpu.sync_copy(dst1d.at[pl.ds(ebase + j * CH, CH)], didx_cur)
        pltpu.async_copy(ones_v, deg_acc.at[didx_cur], sem, add=True).wait()

    pltpu.sync_copy(xids.at[pl.ds(wid * RPT, RPT)], xidx)
    for j in range(RPT // 64):
        pltpu.async_copy(emb.at[xidx.at[pl.ds(j * 64, 64)]], erows, sem).wait()
        pltpu.sync_copy(erows, ex_out.at[pl.ds(wid * RPT + j * 64, 64)])

    plsc.subcore_barrier()
    pltpu.sync_copy(deg_acc.at[pl.ds(row0, ROWS_PER_SC_TILE)],
                    deg_out.at[cid, pl.ds(row0, ROWS_PER_SC_TILE)])


def _sc_edge_agg_body(hs, src1d, dst1d, z128, agg_out, sidx0, sidx1, sidx2,
                      didx0, didx1, didx2, rows0, rows1, acc, sem0, sem1,
                      isem0, isem1, isem2, ssem0, ssem1):
    cid = lax.axis_index("c")
    sid = lax.axis_index("s")
    wid = sid * NC + cid

    row0 = sid * ROWS_PER_SC_TILE
    pltpu.sync_copy(z128.at[pl.ds(row0, ROWS_PER_SC_TILE)],
                    acc.at[pl.ds(row0, ROWS_PER_SC_TILE)])
    plsc.subcore_barrier()

    ebase = wid * EPT_PAD
    sidx = (sidx0, sidx1, sidx2)
    didx = (didx0, didx1, didx2)
    rows = (rows0, rows1)
    gsems = (sem0, sem1)
    isems = (isem0, isem1, isem2)
    ssems = (ssem0, ssem1)

    pltpu.sync_copy(src1d.at[pl.ds(ebase, CH)], sidx[0])
    pltpu.sync_copy(dst1d.at[pl.ds(ebase, CH)], didx[0])
    gather = [None, None]
    idxs = [None, None, None]
    idxd = [None, None, None]
    scat = [None, None]
    gather[0] = pltpu.async_copy(hs.at[sidx[0]], rows[0], gsems[0])
    if NCHUNK > 1:
        idxs[1] = pltpu.async_copy(src1d.at[pl.ds(ebase + CH, CH)], sidx[1],
                                   isems[1])
        idxd[1] = pltpu.async_copy(dst1d.at[pl.ds(ebase + CH, CH)], didx[1],
                                   isems[1])

    for j in range(NCHUNK):
        b2 = j % 2
        n2 = (j + 1) % 2
        b3 = j % 3
        n3 = (j + 1) % 3
        if j + 1 < NCHUNK:
            idxs[n3].wait()
            idxd[n3].wait()
            if scat[n2] is not None:
                scat[n2].wait()
            gather[n2] = pltpu.async_copy(hs.at[sidx[n3]], rows[n2], gsems[n2])
        gather[b2].wait()
        scat[b2] = pltpu.async_copy(rows[b2], acc.at[didx[b3]], ssems[b2],
                                    add=True)
        if j + 2 < NCHUNK:
            off = ebase + (j + 2) * CH
            p3 = (j + 2) % 3
            idxs[p3] = pltpu.async_copy(src1d.at[pl.ds(off, CH)], sidx[p3],
                                        isems[p3])
            idxd[p3] = pltpu.async_copy(dst1d.at[pl.ds(off, CH)], didx[p3],
                                        isems[p3])
    scat[(NCHUNK - 1) % 2].wait()
    if scat[NCHUNK % 2] is not None:
        scat[NCHUNK % 2].wait()

    plsc.subcore_barrier()
    pltpu.sync_copy(acc.at[pl.ds(row0, ROWS_PER_SC_TILE)],
                    agg_out.at[cid, pl.ds(row0, ROWS_PER_SC_TILE)])


@functools.cache
def _sc_kernels():
    mesh = plsc.VectorSubcoreMesh(core_axis_name="c", subcore_axis_name="s",
                                  num_cores=NC, num_subcores=NS)
    deg_gather = pl.kernel(
        _sc_deg_gather_body,
        out_type=(
            jax.ShapeDtypeStruct((NC, NPAD, D), jnp.float32),
            jax.ShapeDtypeStruct((NPAD, D), jnp.float32),
        ),
        mesh=mesh,
        scratch_types=[
            pltpu.VMEM((CH,), jnp.int32),
            pltpu.VMEM((RPT,), jnp.int32),
            pltpu.VMEM((CH, D), jnp.float32),
            pltpu.VMEM((64, D), jnp.float32),
            pltpu.VMEM_SHARED((NPAD, D), jnp.float32),
            pltpu.SemaphoreType.DMA,
        ],
    )
    edge_agg = pl.kernel(
        _sc_edge_agg_body,
        out_type=jax.ShapeDtypeStruct((NC, NPAD, D), jnp.float32),
        mesh=mesh,
        scratch_types=[
            pltpu.VMEM((CH,), jnp.int32),
            pltpu.VMEM((CH,), jnp.int32),
            pltpu.VMEM((CH,), jnp.int32),
            pltpu.VMEM((CH,), jnp.int32),
            pltpu.VMEM((CH,), jnp.int32),
            pltpu.VMEM((CH,), jnp.int32),
            pltpu.VMEM((CH, D), jnp.float32),
            pltpu.VMEM((CH, D), jnp.float32),
            pltpu.VMEM_SHARED((NPAD, D), jnp.float32),
            pltpu.SemaphoreType.DMA,
            pltpu.SemaphoreType.DMA,
            pltpu.SemaphoreType.DMA,
            pltpu.SemaphoreType.DMA,
            pltpu.SemaphoreType.DMA,
            pltpu.SemaphoreType.DMA,
            pltpu.SemaphoreType.DMA,
        ],
    )
    return deg_gather, edge_agg


BR = 512
GRID = NPAD // BR


def _dinv_from(degp):
    deg = degp[0, :, 0] + degp[1, :, 0] + 1.0
    return lax.rsqrt(deg)[:, None]


def _tc1_body(ex_ref, degp_ref, w0_ref, wskip_ref, bskip_ref, hs1_ref, skip_ref):
    ex = ex_ref[...]
    dinv = _dinv_from(degp_ref[...])
    h2 = jnp.dot(ex, w0_ref[...], preferred_element_type=jnp.float32)
    hs1_ref[...] = h2 * dinv
    skip_ref[...] = (jnp.dot(ex, wskip_ref[...], preferred_element_type=jnp.float32)
                     + bskip_ref[...])


def _layer_norm(h, g, b):
    mu = jnp.mean(h, axis=-1, keepdims=True)
    var = jnp.mean((h - mu) * (h - mu), axis=-1, keepdims=True)
    return (h - mu) * lax.rsqrt(var + 1e-5) * g + b


def _tc2_body(aggp_ref, hs1_ref, degp_ref, skip_ref, w1_ref, b0_ref,
              g0_ref, be0_ref, hs2_ref):
    dinv = _dinv_from(degp_ref[...])
    aggp = aggp_ref[...]
    out = (aggp[0] + aggp[1] + hs1_ref[...]) * dinv + b0_ref[...]
    h = jax.nn.relu(_layer_norm(out, g0_ref[...], be0_ref[...]))
    u = skip_ref[...] + h
    h2b = jnp.dot(u, w1_ref[...], preferred_element_type=jnp.float32)
    hs2_ref[...] = h2b * dinv


def _tc3_body(aggp_ref, hs2_ref, degp_ref, b1_ref, g1_ref, be1_ref, ptr_ref,
              h2_ref, g_ref):
    i = pl.program_id(0)
    dinv = _dinv_from(degp_ref[...])
    aggp = aggp_ref[...]
    out = (aggp[0] + aggp[1] + hs2_ref[...]) * dinv + b1_ref[...]
    h2 = jax.nn.relu(_layer_norm(out, g1_ref[...], be1_ref[...]))
    h2_ref[...] = h2

    rows = lax.broadcasted_iota(jnp.int32, (NB, BR), 1) + i * BR
    lo = jnp.stack([ptr_ref[b] for b in range(NB)])[:, None]
    hi = jnp.stack([ptr_ref[b + 1] for b in range(NB)])[:, None]
    cnt = jnp.maximum((hi - lo).astype(jnp.float32), 1.0)
    p = jnp.where((rows >= lo) & (rows < hi), 1.0, 0.0) / cnt
    gp = jnp.dot(p, h2, preferred_element_type=jnp.float32)

    @pl.when(i == 0)
    def _():
        g_ref[...] = jnp.zeros_like(g_ref)

    g_ref[...] += gp


def _row_spec(i_map=lambda i: (i, 0)):
    return pl.BlockSpec((BR, D), i_map)


_full128 = pl.BlockSpec((D, D), lambda i: (0, 0))
_vec128 = pl.BlockSpec((1, D), lambda i: (0, 0))
_degp_spec = pl.BlockSpec((NC, BR, D), lambda i: (0, i, 0))
_aggp_spec = pl.BlockSpec((NC, BR, D), lambda i: (0, i, 0))


def _tc1(ex, degp, w0, wskip, bskip):
    return pl.pallas_call(
        _tc1_body,
        grid=(GRID,),
        in_specs=[_row_spec(), _degp_spec, _full128, _full128, _vec128],
        out_specs=[_row_spec(), _row_spec()],
        out_shape=[jax.ShapeDtypeStruct((NPAD, D), jnp.float32)] * 2,
    )(ex, degp, w0, wskip, bskip)


def _tc2(aggp, hs1, degp, skip, w1, b0, g0, be0):
    return pl.pallas_call(
        _tc2_body,
        grid=(GRID,),
        in_specs=[_aggp_spec, _row_spec(), _degp_spec, _row_spec(), _full128,
                  _vec128, _vec128, _vec128],
        out_specs=_row_spec(),
        out_shape=jax.ShapeDtypeStruct((NPAD, D), jnp.float32),
    )(aggp, hs1, degp, skip, w1, b0, g0, be0)


def _tc3(aggp, hs2, degp, b1, g1, be1, ptr):
    return pl.pallas_call(
        _tc3_body,
        grid=(GRID,),
        in_specs=[_aggp_spec, _row_spec(), _degp_spec, _vec128, _vec128,
                  _vec128, pl.BlockSpec(memory_space=pltpu.SMEM)],
        out_specs=[_row_spec(), pl.BlockSpec((NB, D), lambda i: (0, 0))],
        out_shape=[jax.ShapeDtypeStruct((NPAD, D), jnp.float32),
                   jax.ShapeDtypeStruct((NB, D), jnp.float32)],
    )(aggp, hs2, degp, b1, g1, be1, ptr)


def kernel(x, edge_index_x, ptr_x, y, edge_index_y, ptr_y, emb,
           W_skip, b_skip, W0, b0, W1, b1, ln0_g, ln0_b, ln1_g, ln1_b):
    src = edge_index_x[0]
    dst = edge_index_x[1]
    pad_e = EPAD - E
    src1d = jnp.concatenate([src, jnp.zeros((pad_e,), jnp.int32)])
    dst1d = jnp.concatenate([dst, jnp.full((pad_e,), DUMP_ROW, jnp.int32)])
    xids = jnp.concatenate([x, jnp.zeros((NPAD - N,), jnp.int32)])

    _sc_deg_gather, _sc_edge_agg = _sc_kernels()
    z128 = jnp.zeros((NPAD, D), jnp.float32)
    ones_h = jnp.zeros((CH, D), jnp.float32).at[:, 0].set(1.0)
    degp, ex = _sc_deg_gather(dst1d, xids, emb, z128, ones_h)

    hs1, skip = _tc1(ex, degp, W0, W_skip, b_skip.reshape(1, D))
    agg1 = _sc_edge_agg(hs1, src1d, dst1d, z128)
    hs2 = _tc2(agg1, hs1, degp, skip, W1, b0.reshape(1, D),
               ln0_g.reshape(1, D), ln0_b.reshape(1, D))
    agg2 = _sc_edge_agg(hs2, src1d, dst1d, z128)
    h2pad, g = _tc3(agg2, hs2, degp, b1.reshape(1, D),
                    ln1_g.reshape(1, D), ln1_b.reshape(1, D), ptr_x)
    return (h2pad[:N], g)
